# Initial kernel scaffold; baseline (speedup 1.0000x reference)
import jax
import jax.numpy as jnp
from jax.experimental import pallas as pl


def kernel(x, edge_index, batch, W1, b1, W2, b2, W3, b3, Wfc, bfc):
    raise NotImplementedError("write your pallas kernel here")


# speedup vs baseline: 13.6736x; 13.6736x over previous
import functools

import jax
import jax.numpy as jnp
from jax import lax
from jax.experimental import pallas as pl
from jax.experimental.pallas import tpu as pltpu
from jax.experimental.pallas import tpu_sc as plsc

_N = 10000
_E = 320000
_G = 64

_NC = 2
_NS = 16
_NW = _NC * _NS
_EPW = _E // _NW
_K = 80
_NCHUNK = _EPW // _K
_NP = 10240
_RPS = _NP // _NS
_ZR = 128
_DEGW = 16

@functools.lru_cache(maxsize=None)
def _sc_mesh():
    return plsc.VectorSubcoreMesh(core_axis_name="c", subcore_axis_name="s",
                                  num_cores=_NC, num_subcores=_NS)


def _zero_fill(zrow_v, d):
    z16 = jnp.zeros((16,), jnp.float32)
    for r in range(_ZR):
        for cc in range(d // 16):
            zrow_v[r, pl.ds(cc * 16, 16)] = z16


def _zero_accum(zrow_v, accum_sh, s):
    for j in range(_RPS // _ZR):
        pltpu.sync_copy(zrow_v, accum_sh.at[pl.ds(s * _RPS + j * _ZR, _ZR)])


def _deg_body(dst_hbm, out_hbm, didx_v, ones_v, zrow_v, accum_sh):
    c = lax.axis_index("c")
    s = lax.axis_index("s")
    one16 = jnp.ones((16,), jnp.float32)
    for r in range(_K):
        ones_v[r, :] = one16
    _zero_fill(zrow_v, _DEGW)
    _zero_accum(zrow_v, accum_sh, s)
    plsc.subcore_barrier()
    base = (s * _NC + c) * _EPW

    def step(i, carry):
        pltpu.sync_copy(dst_hbm.at[pl.ds(base + i * _K, _K)], didx_v)
        pltpu.sync_copy(ones_v, accum_sh.at[didx_v], add=True)
        return carry

    lax.fori_loop(0, _NCHUNK, step, 0)
    plsc.subcore_barrier()
    pltpu.sync_copy(accum_sh.at[pl.ds(s * _RPS, _RPS)],
                    out_hbm.at[c, pl.ds(s * _RPS, _RPS)])


@functools.lru_cache(maxsize=None)
def _deg_call():
    return pl.kernel(
        _deg_body,
        out_type=jax.ShapeDtypeStruct((_NC, _NP, _DEGW), jnp.float32),
        mesh=_sc_mesh(),
        compiler_params=pltpu.CompilerParams(use_tc_tiling_on_sc=False),
        scratch_types=[
            pltpu.VMEM((_K,), jnp.int32),
            pltpu.VMEM((_K, _DEGW), jnp.float32),
            pltpu.VMEM((_ZR, _DEGW), jnp.float32),
            pltpu.VMEM_SHARED((_NP, _DEGW), jnp.float32),
        ],
    )


def _edge_body(d, g_hbm, src_hbm, dst_hbm, out_hbm,
               sidx_v, didx_v, rows_v, zrow_v, sem, accum_sh):
    c = lax.axis_index("c")
    s = lax.axis_index("s")
    _zero_fill(zrow_v, d)
    _zero_accum(zrow_v, accum_sh, s)
    plsc.subcore_barrier()
    base = (s * _NC + c) * _EPW

    def step(i, carry):
        pltpu.sync_copy(src_hbm.at[pl.ds(base + i * _K, _K)], sidx_v)
        pltpu.sync_copy(dst_hbm.at[pl.ds(base + i * _K, _K)], didx_v)
        pltpu.async_copy(g_hbm.at[sidx_v], rows_v, sem).wait()
        pltpu.sync_copy(rows_v, accum_sh.at[didx_v], add=True)
        return carry

    lax.fori_loop(0, _NCHUNK, step, 0)
    plsc.subcore_barrier()
    pltpu.sync_copy(accum_sh.at[pl.ds(s * _RPS, _RPS)],
                    out_hbm.at[c, pl.ds(s * _RPS, _RPS)])


@functools.lru_cache(maxsize=None)
def _make_edge_call(d):
    return pl.kernel(
        functools.partial(_edge_body, d),
        out_type=jax.ShapeDtypeStruct((_NC, _NP, d), jnp.float32),
        mesh=_sc_mesh(),
        compiler_params=pltpu.CompilerParams(use_tc_tiling_on_sc=False),
        scratch_types=[
            pltpu.VMEM((_K,), jnp.int32),
            pltpu.VMEM((_K,), jnp.int32),
            pltpu.VMEM((_K, d), jnp.float32),
            pltpu.VMEM((_ZR, d), jnp.float32),
            pltpu.SemaphoreType.DMA,
            pltpu.VMEM_SHARED((_NP, d), jnp.float32),
        ],
    )


_HI = lax.Precision.HIGHEST


def _b1_body(x_ref, w1_ref, degp_ref, g1_ref, dinv_ref):
    deg = (degp_ref[0, :_N, 0:1] + degp_ref[1, :_N, 0:1]) + 1.0
    dinv = lax.rsqrt(deg)
    h = jnp.dot(x_ref[...], w1_ref[...],
                preferred_element_type=jnp.float32, precision=_HI)
    g1_ref[...] = h * dinv
    dinv_ref[...] = dinv


def _b1_call(x, w1, degp):
    return pl.pallas_call(
        _b1_body,
        out_shape=(
            jax.ShapeDtypeStruct((_N, 64), jnp.float32),
            jax.ShapeDtypeStruct((_N, 1), jnp.float32),
        ),
    )(x, w1, degp)


def _mid_body(p_ref, g_ref, dinv_ref, b_ref, w_ref, out_ref):
    svals = p_ref[0, :_N, :] + p_ref[1, :_N, :] + g_ref[...]
    h = jnp.maximum(svals * dinv_ref[...] + b_ref[...], 0.0)
    out_ref[...] = jnp.dot(h, w_ref[...],
                           preferred_element_type=jnp.float32,
                           precision=_HI) * dinv_ref[...]


def _mid_call(p, g, dinv, b, w):
    return pl.pallas_call(
        _mid_body,
        out_shape=jax.ShapeDtypeStruct((_N, w.shape[1]), jnp.float32),
    )(p, g, dinv, b, w)


def _final_body(p_ref, g_ref, dinv_ref, b_ref, batch_ref, wfc_ref, bfc_ref,
                out_ref):
    svals = p_ref[0, :_N, :] + p_ref[1, :_N, :] + g_ref[...]
    h = jnp.maximum(svals * dinv_ref[...] + b_ref[...], 0.0)
    gid = lax.broadcasted_iota(jnp.int32, (_G, _N), 0)
    onehot_t = (batch_ref[...] == gid).astype(jnp.float32)
    sums = jnp.dot(onehot_t, h,
                   preferred_element_type=jnp.float32, precision=_HI)
    cnt = jnp.sum(onehot_t, axis=1)[:, None]
    pooled = sums / jnp.maximum(cnt, 1.0)
    logit = jnp.dot(pooled, wfc_ref[...],
                    preferred_element_type=jnp.float32,
                    precision=_HI) + bfc_ref[...]
    out_ref[...] = jax.nn.sigmoid(logit)


def _final_call(p, g, dinv, b, batch2d, wfc, bfc):
    return pl.pallas_call(
        _final_body,
        out_shape=jax.ShapeDtypeStruct((_G, 1), jnp.float32),
    )(p, g, dinv, b, batch2d, wfc, bfc)


def kernel(x, edge_index, batch, W1, b1, W2, b2, W3, b3, Wfc, bfc):
    src = edge_index[0].astype(jnp.int32)
    dst = edge_index[1].astype(jnp.int32)
    batch2d = batch.astype(jnp.int32).reshape(1, _N)

    degp = _deg_call()(dst)
    g1, dinv = _b1_call(x, W1, degp)
    p1 = _make_edge_call(64)(g1, src, dst)
    g2 = _mid_call(p1, g1, dinv, b1.reshape(1, -1), W2)
    p2 = _make_edge_call(32)(g2, src, dst)
    g3 = _mid_call(p2, g2, dinv, b2.reshape(1, -1), W3)
    p3 = _make_edge_call(16)(g3, src, dst)
    out = _final_call(p3, g3, dinv, b3.reshape(1, -1), batch2d,
                      Wfc, bfc.reshape(1, 1))
    return out


# speedup vs baseline: 37.2266x; 2.7225x over previous
import functools

import jax
import jax.numpy as jnp
from jax import lax
from jax.experimental import pallas as pl
from jax.experimental.pallas import tpu as pltpu
from jax.experimental.pallas import tpu_sc as plsc

_N = 10000
_E = 320000
_G = 64

_NC = 2
_NS = 16
_NW = _NC * _NS
_EPW = _E // _NW
_K = 100
_NCHUNK = _EPW // _K
_GRP = 4
_NP = 10240
_RPS = _NP // _NS
_ZR = 128
_DEGW = 16

@functools.lru_cache(maxsize=None)
def _sc_mesh():
    return plsc.VectorSubcoreMesh(core_axis_name="c", subcore_axis_name="s",
                                  num_cores=_NC, num_subcores=_NS)


def _zero_fill(zrow_v, d):
    z16 = jnp.zeros((16,), jnp.float32)
    for r in range(_ZR):
        for cc in range(d // 16):
            zrow_v[r, pl.ds(cc * 16, 16)] = z16


def _zero_accum(zrow_v, accum_sh, s):
    for j in range(_RPS // _ZR):
        pltpu.sync_copy(zrow_v, accum_sh.at[pl.ds(s * _RPS + j * _ZR, _ZR)])


def _deg_body(dst_hbm, out_hbm, didx_v, ones_v, zrow_v, ssem, accum_sh):
    c = lax.axis_index("c")
    s = lax.axis_index("s")
    one16 = jnp.ones((16,), jnp.float32)
    for r in range(_K):
        ones_v[r, :] = one16
    _zero_fill(zrow_v, _DEGW)
    _zero_accum(zrow_v, accum_sh, s)
    plsc.subcore_barrier()
    rbase = (s * _NC + c) * _NCHUNK
    pltpu.sync_copy(dst_hbm.at[pl.ds(rbase, _NCHUNK)], didx_v)

    def group(j, carry):
        descs = [pltpu.async_copy(ones_v, accum_sh.at[didx_v.at[_GRP * j + b]],
                                  ssem, add=True)
                 for b in range(_GRP)]
        for dsc in descs:
            dsc.wait()
        return carry

    lax.fori_loop(0, _NCHUNK // _GRP, group, 0)
    plsc.subcore_barrier()
    pltpu.sync_copy(accum_sh.at[pl.ds(s * _RPS, _RPS)],
                    out_hbm.at[c, pl.ds(s * _RPS, _RPS)])


@functools.lru_cache(maxsize=None)
def _deg_call():
    return pl.kernel(
        _deg_body,
        out_type=jax.ShapeDtypeStruct((_NC, _NP, _DEGW), jnp.float32),
        mesh=_sc_mesh(),
        compiler_params=pltpu.CompilerParams(use_tc_tiling_on_sc=False),
        scratch_types=[
            pltpu.VMEM((_NCHUNK, _K), jnp.int32),
            pltpu.VMEM((_K, _DEGW), jnp.float32),
            pltpu.VMEM((_ZR, _DEGW), jnp.float32),
            pltpu.SemaphoreType.DMA,
            pltpu.VMEM_SHARED((_NP, _DEGW), jnp.float32),
        ],
    )


def _edge_body(d, g_hbm, src_hbm, dst_hbm, out_hbm,
               sidx_v, didx_v, rows_v, zrow_v, gsem, ssem, accum_sh):
    c = lax.axis_index("c")
    s = lax.axis_index("s")
    _zero_fill(zrow_v, d)
    _zero_accum(zrow_v, accum_sh, s)
    plsc.subcore_barrier()
    rbase = (s * _NC + c) * _NCHUNK
    pltpu.sync_copy(src_hbm.at[pl.ds(rbase, _NCHUNK)], sidx_v)
    pltpu.sync_copy(dst_hbm.at[pl.ds(rbase, _NCHUNK)], didx_v)

    def group(j, carry):
        gds = [pltpu.async_copy(g_hbm.at[sidx_v.at[_GRP * j + b]],
                                rows_v.at[b], gsem)
               for b in range(_GRP)]
        for dsc in gds:
            dsc.wait()
        sds = [pltpu.async_copy(rows_v.at[b],
                                accum_sh.at[didx_v.at[_GRP * j + b]],
                                ssem, add=True)
               for b in range(_GRP)]
        for dsc in sds:
            dsc.wait()
        return carry

    lax.fori_loop(0, _NCHUNK // _GRP, group, 0)
    plsc.subcore_barrier()
    pltpu.sync_copy(accum_sh.at[pl.ds(s * _RPS, _RPS)],
                    out_hbm.at[c, pl.ds(s * _RPS, _RPS)])


@functools.lru_cache(maxsize=None)
def _make_edge_call(d):
    return pl.kernel(
        functools.partial(_edge_body, d),
        out_type=jax.ShapeDtypeStruct((_NC, _NP, d), jnp.float32),
        mesh=_sc_mesh(),
        compiler_params=pltpu.CompilerParams(use_tc_tiling_on_sc=False),
        scratch_types=[
            pltpu.VMEM((_NCHUNK, _K), jnp.int32),
            pltpu.VMEM((_NCHUNK, _K), jnp.int32),
            pltpu.VMEM((_GRP, _K, d), jnp.float32),
            pltpu.VMEM((_ZR, d), jnp.float32),
            pltpu.SemaphoreType.DMA,
            pltpu.SemaphoreType.DMA,
            pltpu.VMEM_SHARED((_NP, d), jnp.float32),
        ],
    )


_HI = lax.Precision.HIGHEST


def _b1_body(x_ref, w1_ref, degp_ref, g1_ref, dinv_ref):
    deg = (degp_ref[0, :_N, 0:1] + degp_ref[1, :_N, 0:1]) + 1.0
    dinv = lax.rsqrt(deg)
    h = jnp.dot(x_ref[...], w1_ref[...],
                preferred_element_type=jnp.float32, precision=_HI)
    g1_ref[...] = h * dinv
    dinv_ref[...] = dinv


def _b1_call(x, w1, degp):
    return pl.pallas_call(
        _b1_body,
        out_shape=(
            jax.ShapeDtypeStruct((_N, 64), jnp.float32),
            jax.ShapeDtypeStruct((_N, 1), jnp.float32),
        ),
    )(x, w1, degp)


def _mid_body(p_ref, g_ref, dinv_ref, b_ref, w_ref, out_ref):
    svals = p_ref[0, :_N, :] + p_ref[1, :_N, :] + g_ref[...]
    h = jnp.maximum(svals * dinv_ref[...] + b_ref[...], 0.0)
    out_ref[...] = jnp.dot(h, w_ref[...],
                           preferred_element_type=jnp.float32,
                           precision=_HI) * dinv_ref[...]


def _mid_call(p, g, dinv, b, w):
    return pl.pallas_call(
        _mid_body,
        out_shape=jax.ShapeDtypeStruct((_N, w.shape[1]), jnp.float32),
    )(p, g, dinv, b, w)


def _final_body(p_ref, g_ref, dinv_ref, b_ref, batch_ref, wfc_ref, bfc_ref,
                out_ref):
    svals = p_ref[0, :_N, :] + p_ref[1, :_N, :] + g_ref[...]
    h = jnp.maximum(svals * dinv_ref[...] + b_ref[...], 0.0)
    gid = lax.broadcasted_iota(jnp.int32, (_G, _N), 0)
    onehot_t = (batch_ref[...] == gid).astype(jnp.float32)
    sums = jnp.dot(onehot_t, h,
                   preferred_element_type=jnp.float32, precision=_HI)
    cnt = jnp.sum(onehot_t, axis=1)[:, None]
    pooled = sums / jnp.maximum(cnt, 1.0)
    logit = jnp.dot(pooled, wfc_ref[...],
                    preferred_element_type=jnp.float32,
                    precision=_HI) + bfc_ref[...]
    out_ref[...] = jax.nn.sigmoid(logit)


def _final_call(p, g, dinv, b, batch2d, wfc, bfc):
    return pl.pallas_call(
        _final_body,
        out_shape=jax.ShapeDtypeStruct((_G, 1), jnp.float32),
    )(p, g, dinv, b, batch2d, wfc, bfc)


def kernel(x, edge_index, batch, W1, b1, W2, b2, W3, b3, Wfc, bfc):
    src = edge_index[0].astype(jnp.int32).reshape(_E // _K, _K)
    dst = edge_index[1].astype(jnp.int32).reshape(_E // _K, _K)
    batch2d = batch.astype(jnp.int32).reshape(1, _N)

    degp = _deg_call()(dst)
    g1, dinv = _b1_call(x, W1, degp)
    p1 = _make_edge_call(64)(g1, src, dst)
    g2 = _mid_call(p1, g1, dinv, b1.reshape(1, -1), W2)
    p2 = _make_edge_call(32)(g2, src, dst)
    g3 = _mid_call(p2, g2, dinv, b2.reshape(1, -1), W3)
    p3 = _make_edge_call(16)(g3, src, dst)
    out = _final_call(p3, g3, dinv, b3.reshape(1, -1), batch2d,
                      Wfc, bfc.reshape(1, 1))
    return out


# speedup vs baseline: 42.2165x; 1.1340x over previous
import functools

import jax
import jax.numpy as jnp
from jax import lax
from jax.experimental import pallas as pl
from jax.experimental.pallas import tpu as pltpu
from jax.experimental.pallas import tpu_sc as plsc

_N = 10000
_E = 320000
_G = 64

_NC = 2
_NS = 16
_NW = _NC * _NS
_EPW = _E // _NW
_K = 50
_NCHUNK = _EPW // _K
_GRP = 5
_NG = _NCHUNK // _GRP
_NP = 10240
_RPS = _NP // _NS
_ZR = 64
_DEGW = 16


@functools.lru_cache(maxsize=None)
def _sc_mesh():
    return plsc.VectorSubcoreMesh(core_axis_name="c", subcore_axis_name="s",
                                  num_cores=_NC, num_subcores=_NS)


def _zero_fill(zrow_v, d):
    z16 = jnp.zeros((16,), jnp.float32)
    for r in range(_ZR):
        for cc in range(d // 16):
            zrow_v[r, pl.ds(cc * 16, 16)] = z16


def _zero_accum(zrow_v, accum_sh, s):
    for j in range(_RPS // _ZR):
        pltpu.sync_copy(zrow_v, accum_sh.at[pl.ds(s * _RPS + j * _ZR, _ZR)])


def _deg_body(dst_hbm, out_hbm, didx_v, ones_v, zrow_v, ssem, accum_sh):
    c = lax.axis_index("c")
    s = lax.axis_index("s")
    one16 = jnp.ones((16,), jnp.float32)
    for r in range(_K):
        ones_v[r, :] = one16
    _zero_fill(zrow_v, _DEGW)
    _zero_accum(zrow_v, accum_sh, s)
    plsc.subcore_barrier()
    rbase = (s * _NC + c) * _NCHUNK
    pltpu.sync_copy(dst_hbm.at[pl.ds(rbase, _NCHUNK)], didx_v)

    def group(j, carry):
        descs = [pltpu.async_copy(ones_v, accum_sh.at[didx_v.at[_GRP * j + b]],
                                  ssem, add=True)
                 for b in range(_GRP)]
        for dsc in descs:
            dsc.wait()
        return carry

    lax.fori_loop(0, _NG, group, 0)
    plsc.subcore_barrier()
    pltpu.sync_copy(accum_sh.at[pl.ds(s * _RPS, _RPS)],
                    out_hbm.at[c, pl.ds(s * _RPS, _RPS)])


@functools.lru_cache(maxsize=None)
def _deg_call():
    return pl.kernel(
        _deg_body,
        out_type=jax.ShapeDtypeStruct((_NC, _NP, _DEGW), jnp.float32),
        mesh=_sc_mesh(),
        compiler_params=pltpu.CompilerParams(use_tc_tiling_on_sc=False),
        scratch_types=[
            pltpu.VMEM((_NCHUNK, _K), jnp.int32),
            pltpu.VMEM((_K, _DEGW), jnp.float32),
            pltpu.VMEM((_ZR, _DEGW), jnp.float32),
            pltpu.SemaphoreType.DMA,
            pltpu.VMEM_SHARED((_NP, _DEGW), jnp.float32),
        ],
    )


def _edge_body(d, g_hbm, src_hbm, dst_hbm, out_hbm,
               sidx_v, didx_v, rows_v, zrow_v, gsem0, gsem1, ssem0, ssem1,
               accum_sh):
    c = lax.axis_index("c")
    s = lax.axis_index("s")
    _zero_fill(zrow_v, d)
    _zero_accum(zrow_v, accum_sh, s)
    plsc.subcore_barrier()
    rbase = (s * _NC + c) * _NCHUNK
    pltpu.sync_copy(src_hbm.at[pl.ds(rbase, _NCHUNK)], sidx_v)
    pltpu.sync_copy(dst_hbm.at[pl.ds(rbase, _NCHUNK)], didx_v)

    def fire_gathers(g, bank, sem):
        for b in range(_GRP):
            pltpu.async_copy(g_hbm.at[sidx_v.at[_GRP * g + b]],
                             rows_v.at[bank, b], sem)

    def drain_gathers(bank, sem):
        for b in range(_GRP):
            pltpu.make_async_copy(g_hbm.at[sidx_v.at[b]],
                                  rows_v.at[bank, b], sem).wait()

    def fire_scatters(g, bank, sem):
        return [pltpu.async_copy(rows_v.at[bank, b],
                                 accum_sh.at[didx_v.at[_GRP * g + b]],
                                 sem, add=True)
                for b in range(_GRP)]

    fire_gathers(0, 0, gsem0)
    fire_gathers(1, 1, gsem1)

    def pipe(j, carry):
        drain_gathers(0, gsem0)
        s_a = fire_scatters(2 * j, 0, ssem0)
        drain_gathers(1, gsem1)
        for dsc in s_a:
            dsc.wait()

        @pl.when(j + 1 < _NG // 2)
        def _():
            fire_gathers(2 * j + 2, 0, gsem0)

        s_b = fire_scatters(2 * j + 1, 1, ssem1)
        for dsc in s_b:
            dsc.wait()

        @pl.when(j + 1 < _NG // 2)
        def _():
            fire_gathers(2 * j + 3, 1, gsem1)

        return carry

    lax.fori_loop(0, _NG // 2, pipe, 0)
    plsc.subcore_barrier()
    pltpu.sync_copy(accum_sh.at[pl.ds(s * _RPS, _RPS)],
                    out_hbm.at[c, pl.ds(s * _RPS, _RPS)])


@functools.lru_cache(maxsize=None)
def _make_edge_call(d):
    return pl.kernel(
        functools.partial(_edge_body, d),
        out_type=jax.ShapeDtypeStruct((_NC, _NP, d), jnp.float32),
        mesh=_sc_mesh(),
        compiler_params=pltpu.CompilerParams(use_tc_tiling_on_sc=False),
        scratch_types=[
            pltpu.VMEM((_NCHUNK, _K), jnp.int32),
            pltpu.VMEM((_NCHUNK, _K), jnp.int32),
            pltpu.VMEM((2, _GRP, _K, d), jnp.float32),
            pltpu.VMEM((_ZR, d), jnp.float32),
            pltpu.SemaphoreType.DMA,
            pltpu.SemaphoreType.DMA,
            pltpu.SemaphoreType.DMA,
            pltpu.SemaphoreType.DMA,
            pltpu.VMEM_SHARED((_NP, d), jnp.float32),
        ],
    )


_HI = lax.Precision.HIGHEST


def _b1_body(x_ref, w1_ref, degp_ref, g1_ref, dinv_ref):
    deg = (degp_ref[0, :_N, 0:1] + degp_ref[1, :_N, 0:1]) + 1.0
    dinv = lax.rsqrt(deg)
    h = jnp.dot(x_ref[...], w1_ref[...],
                preferred_element_type=jnp.float32, precision=_HI)
    g1_ref[...] = h * dinv
    dinv_ref[...] = dinv


def _b1_call(x, w1, degp):
    return pl.pallas_call(
        _b1_body,
        out_shape=(
            jax.ShapeDtypeStruct((_N, 64), jnp.float32),
            jax.ShapeDtypeStruct((_N, 1), jnp.float32),
        ),
    )(x, w1, degp)


def _mid_body(p_ref, g_ref, dinv_ref, b_ref, w_ref, out_ref):
    svals = p_ref[0, :_N, :] + p_ref[1, :_N, :] + g_ref[...]
    h = jnp.maximum(svals * dinv_ref[...] + b_ref[...], 0.0)
    out_ref[...] = jnp.dot(h, w_ref[...],
                           preferred_element_type=jnp.float32,
                           precision=_HI) * dinv_ref[...]


def _mid_call(p, g, dinv, b, w):
    return pl.pallas_call(
        _mid_body,
        out_shape=jax.ShapeDtypeStruct((_N, w.shape[1]), jnp.float32),
    )(p, g, dinv, b, w)


def _final_body(p_ref, g_ref, dinv_ref, b_ref, batch_ref, wfc_ref, bfc_ref,
                out_ref):
    svals = p_ref[0, :_N, :] + p_ref[1, :_N, :] + g_ref[...]
    h = jnp.maximum(svals * dinv_ref[...] + b_ref[...], 0.0)
    gid = lax.broadcasted_iota(jnp.int32, (_G, _N), 0)
    onehot_t = (batch_ref[...] == gid).astype(jnp.float32)
    sums = jnp.dot(onehot_t, h,
                   preferred_element_type=jnp.float32, precision=_HI)
    cnt = jnp.sum(onehot_t, axis=1)[:, None]
    pooled = sums / jnp.maximum(cnt, 1.0)
    logit = jnp.dot(pooled, wfc_ref[...],
                    preferred_element_type=jnp.float32,
                    precision=_HI) + bfc_ref[...]
    out_ref[...] = jax.nn.sigmoid(logit)


def _final_call(p, g, dinv, b, batch2d, wfc, bfc):
    return pl.pallas_call(
        _final_body,
        out_shape=jax.ShapeDtypeStruct((_G, 1), jnp.float32),
    )(p, g, dinv, b, batch2d, wfc, bfc)


def kernel(x, edge_index, batch, W1, b1, W2, b2, W3, b3, Wfc, bfc):
    src = edge_index[0].astype(jnp.int32).reshape(_E // _K, _K)
    dst = edge_index[1].astype(jnp.int32).reshape(_E // _K, _K)
    batch2d = batch.astype(jnp.int32).reshape(1, _N)

    degp = _deg_call()(dst)
    g1, dinv = _b1_call(x, W1, degp)
    p1 = _make_edge_call(64)(g1, src, dst)
    g2 = _mid_call(p1, g1, dinv, b1.reshape(1, -1), W2)
    p2 = _make_edge_call(32)(g2, src, dst)
    g3 = _mid_call(p2, g2, dinv, b2.reshape(1, -1), W3)
    p3 = _make_edge_call(16)(g3, src, dst)
    out = _final_call(p3, g3, dinv, b3.reshape(1, -1), batch2d,
                      Wfc, bfc.reshape(1, 1))
    return out


# speedup vs baseline: 43.5543x; 1.0317x over previous
import functools

import jax
import jax.numpy as jnp
from jax import lax
from jax.experimental import pallas as pl
from jax.experimental.pallas import tpu as pltpu
from jax.experimental.pallas import tpu_sc as plsc

_N = 10000
_E = 320000
_G = 64

_NC = 2
_NS = 16
_NW = _NC * _NS
_EPW = _E // _NW
_K = 125
_NCHUNK = _EPW // _K
_GRP = 2
_NG = _NCHUNK // _GRP
_NP = 10240
_RPS = _NP // _NS
_ZR = 64
_DEGW = 16


@functools.lru_cache(maxsize=None)
def _sc_mesh():
    return plsc.VectorSubcoreMesh(core_axis_name="c", subcore_axis_name="s",
                                  num_cores=_NC, num_subcores=_NS)


def _zero_fill(zrow_v, d):
    z16 = jnp.zeros((16,), jnp.float32)
    for r in range(_ZR):
        for cc in range(d // 16):
            zrow_v[r, pl.ds(cc * 16, 16)] = z16


def _zero_accum(zrow_v, accum_sh, s):
    for j in range(_RPS // _ZR):
        pltpu.sync_copy(zrow_v, accum_sh.at[pl.ds(s * _RPS + j * _ZR, _ZR)])


def _deg_body(dst_hbm, out_hbm, didx_v, ones_v, zrow_v, ssem, accum_sh):
    c = lax.axis_index("c")
    s = lax.axis_index("s")
    one16 = jnp.ones((16,), jnp.float32)
    for r in range(_K):
        ones_v[r, :] = one16
    _zero_fill(zrow_v, _DEGW)
    _zero_accum(zrow_v, accum_sh, s)
    plsc.subcore_barrier()
    rbase = (s * _NC + c) * _NCHUNK
    pltpu.sync_copy(dst_hbm.at[pl.ds(rbase, _NCHUNK)], didx_v)

    def group(j, carry):
        descs = [pltpu.async_copy(ones_v, accum_sh.at[didx_v.at[_GRP * j + b]],
                                  ssem, add=True)
                 for b in range(_GRP)]
        for dsc in descs:
            dsc.wait()
        return carry

    lax.fori_loop(0, _NG, group, 0)
    plsc.subcore_barrier()
    pltpu.sync_copy(accum_sh.at[pl.ds(s * _RPS, _RPS)],
                    out_hbm.at[c, pl.ds(s * _RPS, _RPS)])


@functools.lru_cache(maxsize=None)
def _deg_call():
    return pl.kernel(
        _deg_body,
        out_type=jax.ShapeDtypeStruct((_NC, _NP, _DEGW), jnp.float32),
        mesh=_sc_mesh(),
        compiler_params=pltpu.CompilerParams(use_tc_tiling_on_sc=False),
        scratch_types=[
            pltpu.VMEM((_NCHUNK, _K), jnp.int32),
            pltpu.VMEM((_K, _DEGW), jnp.float32),
            pltpu.VMEM((_ZR, _DEGW), jnp.float32),
            pltpu.SemaphoreType.DMA,
            pltpu.VMEM_SHARED((_NP, _DEGW), jnp.float32),
        ],
    )


def _edge_body(d, g_hbm, src_hbm, dst_hbm, out_hbm,
               sidx_v, didx_v, rows_v, zrow_v, gsem0, gsem1, ssem0, ssem1,
               accum_sh):
    c = lax.axis_index("c")
    s = lax.axis_index("s")
    _zero_fill(zrow_v, d)
    _zero_accum(zrow_v, accum_sh, s)
    plsc.subcore_barrier()
    rbase = (s * _NC + c) * _NCHUNK
    pltpu.sync_copy(src_hbm.at[pl.ds(rbase, _NCHUNK)], sidx_v)
    pltpu.sync_copy(dst_hbm.at[pl.ds(rbase, _NCHUNK)], didx_v)

    def fire_gathers(g, bank, sem):
        for b in range(_GRP):
            pltpu.async_copy(g_hbm.at[sidx_v.at[_GRP * g + b]],
                             rows_v.at[bank, b], sem)

    def drain_gathers(bank, sem):
        for b in range(_GRP):
            pltpu.make_async_copy(g_hbm.at[sidx_v.at[b]],
                                  rows_v.at[bank, b], sem).wait()

    def fire_scatters(g, bank, sem):
        return [pltpu.async_copy(rows_v.at[bank, b],
                                 accum_sh.at[didx_v.at[_GRP * g + b]],
                                 sem, add=True)
                for b in range(_GRP)]

    fire_gathers(0, 0, gsem0)
    fire_gathers(1, 1, gsem1)

    def pipe(j, carry):
        drain_gathers(0, gsem0)
        s_a = fire_scatters(2 * j, 0, ssem0)
        drain_gathers(1, gsem1)
        for dsc in s_a:
            dsc.wait()

        @pl.when(j + 1 < _NG // 2)
        def _():
            fire_gathers(2 * j + 2, 0, gsem0)

        s_b = fire_scatters(2 * j + 1, 1, ssem1)
        for dsc in s_b:
            dsc.wait()

        @pl.when(j + 1 < _NG // 2)
        def _():
            fire_gathers(2 * j + 3, 1, gsem1)

        return carry

    lax.fori_loop(0, _NG // 2, pipe, 0)
    plsc.subcore_barrier()
    pltpu.sync_copy(accum_sh.at[pl.ds(s * _RPS, _RPS)],
                    out_hbm.at[c, pl.ds(s * _RPS, _RPS)])


@functools.lru_cache(maxsize=None)
def _make_edge_call(d):
    return pl.kernel(
        functools.partial(_edge_body, d),
        out_type=jax.ShapeDtypeStruct((_NC, _NP, d), jnp.float32),
        mesh=_sc_mesh(),
        compiler_params=pltpu.CompilerParams(use_tc_tiling_on_sc=False),
        scratch_types=[
            pltpu.VMEM((_NCHUNK, _K), jnp.int32),
            pltpu.VMEM((_NCHUNK, _K), jnp.int32),
            pltpu.VMEM((2, _GRP, _K, d), jnp.float32),
            pltpu.VMEM((_ZR, d), jnp.float32),
            pltpu.SemaphoreType.DMA,
            pltpu.SemaphoreType.DMA,
            pltpu.SemaphoreType.DMA,
            pltpu.SemaphoreType.DMA,
            pltpu.VMEM_SHARED((_NP, d), jnp.float32),
        ],
    )


_HI = lax.Precision.HIGHEST


def _b1_body(x_ref, w1_ref, degp_ref, g1_ref, dinv_ref):
    deg = (degp_ref[0, :_N, 0:1] + degp_ref[1, :_N, 0:1]) + 1.0
    dinv = lax.rsqrt(deg)
    h = jnp.dot(x_ref[...], w1_ref[...],
                preferred_element_type=jnp.float32, precision=_HI)
    g1_ref[...] = h * dinv
    dinv_ref[...] = dinv


def _b1_call(x, w1, degp):
    return pl.pallas_call(
        _b1_body,
        out_shape=(
            jax.ShapeDtypeStruct((_N, 64), jnp.float32),
            jax.ShapeDtypeStruct((_N, 1), jnp.float32),
        ),
    )(x, w1, degp)


def _mid_body(p_ref, g_ref, dinv_ref, b_ref, w_ref, out_ref):
    svals = p_ref[0, :_N, :] + p_ref[1, :_N, :] + g_ref[...]
    h = jnp.maximum(svals * dinv_ref[...] + b_ref[...], 0.0)
    out_ref[...] = jnp.dot(h, w_ref[...],
                           preferred_element_type=jnp.float32,
                           precision=_HI) * dinv_ref[...]


def _mid_call(p, g, dinv, b, w):
    return pl.pallas_call(
        _mid_body,
        out_shape=jax.ShapeDtypeStruct((_N, w.shape[1]), jnp.float32),
    )(p, g, dinv, b, w)


def _final_body(p_ref, g_ref, dinv_ref, b_ref, batch_ref, wfc_ref, bfc_ref,
                out_ref):
    svals = p_ref[0, :_N, :] + p_ref[1, :_N, :] + g_ref[...]
    h = jnp.maximum(svals * dinv_ref[...] + b_ref[...], 0.0)
    gid = lax.broadcasted_iota(jnp.int32, (_G, _N), 0)
    onehot_t = (batch_ref[...] == gid).astype(jnp.float32)
    sums = jnp.dot(onehot_t, h,
                   preferred_element_type=jnp.float32, precision=_HI)
    cnt = jnp.sum(onehot_t, axis=1)[:, None]
    pooled = sums / jnp.maximum(cnt, 1.0)
    logit = jnp.dot(pooled, wfc_ref[...],
                    preferred_element_type=jnp.float32,
                    precision=_HI) + bfc_ref[...]
    out_ref[...] = jax.nn.sigmoid(logit)


def _final_call(p, g, dinv, b, batch2d, wfc, bfc):
    return pl.pallas_call(
        _final_body,
        out_shape=jax.ShapeDtypeStruct((_G, 1), jnp.float32),
    )(p, g, dinv, b, batch2d, wfc, bfc)


def kernel(x, edge_index, batch, W1, b1, W2, b2, W3, b3, Wfc, bfc):
    src = edge_index[0].astype(jnp.int32).reshape(_E // _K, _K)
    dst = edge_index[1].astype(jnp.int32).reshape(_E // _K, _K)
    batch2d = batch.astype(jnp.int32).reshape(1, _N)

    degp = _deg_call()(dst)
    g1, dinv = _b1_call(x, W1, degp)
    p1 = _make_edge_call(64)(g1, src, dst)
    g2 = _mid_call(p1, g1, dinv, b1.reshape(1, -1), W2)
    p2 = _make_edge_call(32)(g2, src, dst)
    g3 = _mid_call(p2, g2, dinv, b2.reshape(1, -1), W3)
    p3 = _make_edge_call(16)(g3, src, dst)
    out = _final_call(p3, g3, dinv, b3.reshape(1, -1), batch2d,
                      Wfc, bfc.reshape(1, 1))
    return out


# speedup vs baseline: 45.5377x; 1.0455x over previous
import functools

import jax
import jax.numpy as jnp
from jax import lax
from jax.experimental import pallas as pl
from jax.experimental.pallas import tpu as pltpu
from jax.experimental.pallas import tpu_sc as plsc

_N = 10000
_E = 320000
_G = 64

_NC = 2
_NS = 16
_NW = _NC * _NS
_EPW = _E // _NW
_K = 125
_NCHUNK = _EPW // _K
_GRP = 4
_NG = _NCHUNK // _GRP
_NP = 10240
_RPS = _NP // _NS
_ZR = 64
_DEGW = 16


@functools.lru_cache(maxsize=None)
def _sc_mesh():
    return plsc.VectorSubcoreMesh(core_axis_name="c", subcore_axis_name="s",
                                  num_cores=_NC, num_subcores=_NS)


def _zero_fill(zrow_v, d):
    z16 = jnp.zeros((16,), jnp.float32)
    for r in range(_ZR):
        for cc in range(d // 16):
            zrow_v[r, pl.ds(cc * 16, 16)] = z16


def _zero_accum(zrow_v, accum_sh, s):
    for j in range(_RPS // _ZR):
        pltpu.sync_copy(zrow_v, accum_sh.at[pl.ds(s * _RPS + j * _ZR, _ZR)])


def _deg_body(dst_hbm, out_hbm, didx_v, ones_v, zrow_v, ssem, accum_sh):
    c = lax.axis_index("c")
    s = lax.axis_index("s")
    one16 = jnp.ones((16,), jnp.float32)
    for r in range(_K):
        ones_v[r, :] = one16
    _zero_fill(zrow_v, _DEGW)
    _zero_accum(zrow_v, accum_sh, s)
    plsc.subcore_barrier()
    rbase = (s * _NC + c) * _NCHUNK
    pltpu.sync_copy(dst_hbm.at[pl.ds(rbase, _NCHUNK)], didx_v)

    def group(j, carry):
        descs = [pltpu.async_copy(ones_v, accum_sh.at[didx_v.at[_GRP * j + b]],
                                  ssem, add=True)
                 for b in range(_GRP)]
        for dsc in descs:
            dsc.wait()
        return carry

    lax.fori_loop(0, _NG, group, 0)
    plsc.subcore_barrier()
    pltpu.sync_copy(accum_sh.at[pl.ds(s * _RPS, _RPS)],
                    out_hbm.at[c, pl.ds(s * _RPS, _RPS)])


@functools.lru_cache(maxsize=None)
def _deg_call():
    return pl.kernel(
        _deg_body,
        out_type=jax.ShapeDtypeStruct((_NC, _NP, _DEGW), jnp.float32),
        mesh=_sc_mesh(),
        compiler_params=pltpu.CompilerParams(use_tc_tiling_on_sc=False),
        scratch_types=[
            pltpu.VMEM((_NCHUNK, _K), jnp.int32),
            pltpu.VMEM((_K, _DEGW), jnp.float32),
            pltpu.VMEM((_ZR, _DEGW), jnp.float32),
            pltpu.SemaphoreType.DMA,
            pltpu.VMEM_SHARED((_NP, _DEGW), jnp.float32),
        ],
    )


def _edge_body(d, g_hbm, src_hbm, dst_hbm, out_hbm,
               sidx_v, didx_v, rows_v, zrow_v, gsem0, gsem1, ssem0, ssem1,
               accum_sh):
    c = lax.axis_index("c")
    s = lax.axis_index("s")
    _zero_fill(zrow_v, d)
    _zero_accum(zrow_v, accum_sh, s)
    plsc.subcore_barrier()
    rbase = (s * _NC + c) * _NCHUNK
    pltpu.sync_copy(src_hbm.at[pl.ds(rbase, _NCHUNK)], sidx_v)
    pltpu.sync_copy(dst_hbm.at[pl.ds(rbase, _NCHUNK)], didx_v)

    def fire_gathers(g, bank, sem):
        for b in range(_GRP):
            pltpu.async_copy(g_hbm.at[sidx_v.at[_GRP * g + b]],
                             rows_v.at[bank, b], sem)

    def drain_gathers(bank, sem):
        for b in range(_GRP):
            pltpu.make_async_copy(g_hbm.at[sidx_v.at[b]],
                                  rows_v.at[bank, b], sem).wait()

    def fire_scatters(g, bank, sem):
        return [pltpu.async_copy(rows_v.at[bank, b],
                                 accum_sh.at[didx_v.at[_GRP * g + b]],
                                 sem, add=True)
                for b in range(_GRP)]

    fire_gathers(0, 0, gsem0)
    fire_gathers(1, 1, gsem1)

    def pipe(j, carry):
        drain_gathers(0, gsem0)
        s_a = fire_scatters(2 * j, 0, ssem0)
        drain_gathers(1, gsem1)
        for dsc in s_a:
            dsc.wait()

        @pl.when(j + 1 < _NG // 2)
        def _():
            fire_gathers(2 * j + 2, 0, gsem0)

        s_b = fire_scatters(2 * j + 1, 1, ssem1)
        for dsc in s_b:
            dsc.wait()

        @pl.when(j + 1 < _NG // 2)
        def _():
            fire_gathers(2 * j + 3, 1, gsem1)

        return carry

    lax.fori_loop(0, _NG // 2, pipe, 0)
    plsc.subcore_barrier()
    pltpu.sync_copy(accum_sh.at[pl.ds(s * _RPS, _RPS)],
                    out_hbm.at[c, pl.ds(s * _RPS, _RPS)])


@functools.lru_cache(maxsize=None)
def _make_edge_call(d):
    return pl.kernel(
        functools.partial(_edge_body, d),
        out_type=jax.ShapeDtypeStruct((_NC, _NP, d), jnp.float32),
        mesh=_sc_mesh(),
        compiler_params=pltpu.CompilerParams(use_tc_tiling_on_sc=False),
        scratch_types=[
            pltpu.VMEM((_NCHUNK, _K), jnp.int32),
            pltpu.VMEM((_NCHUNK, _K), jnp.int32),
            pltpu.VMEM((2, _GRP, _K, d), jnp.float32),
            pltpu.VMEM((_ZR, d), jnp.float32),
            pltpu.SemaphoreType.DMA,
            pltpu.SemaphoreType.DMA,
            pltpu.SemaphoreType.DMA,
            pltpu.SemaphoreType.DMA,
            pltpu.VMEM_SHARED((_NP, d), jnp.float32),
        ],
    )


_HI = lax.Precision.HIGHEST


def _b1_body(x_ref, w1_ref, degp_ref, g1_ref, dinv_ref):
    deg = (degp_ref[0, :_N, 0:1] + degp_ref[1, :_N, 0:1]) + 1.0
    dinv = lax.rsqrt(deg)
    h = jnp.dot(x_ref[...], w1_ref[...],
                preferred_element_type=jnp.float32, precision=_HI)
    g1_ref[...] = h * dinv
    dinv_ref[...] = dinv


def _b1_call(x, w1, degp):
    return pl.pallas_call(
        _b1_body,
        out_shape=(
            jax.ShapeDtypeStruct((_N, 64), jnp.float32),
            jax.ShapeDtypeStruct((_N, 1), jnp.float32),
        ),
    )(x, w1, degp)


def _mid_body(p_ref, g_ref, dinv_ref, b_ref, w_ref, out_ref):
    svals = p_ref[0, :_N, :] + p_ref[1, :_N, :] + g_ref[...]
    h = jnp.maximum(svals * dinv_ref[...] + b_ref[...], 0.0)
    out_ref[...] = jnp.dot(h, w_ref[...],
                           preferred_element_type=jnp.float32,
                           precision=_HI) * dinv_ref[...]


def _mid_call(p, g, dinv, b, w):
    return pl.pallas_call(
        _mid_body,
        out_shape=jax.ShapeDtypeStruct((_N, w.shape[1]), jnp.float32),
    )(p, g, dinv, b, w)


def _final_body(p_ref, g_ref, dinv_ref, b_ref, batch_ref, wfc_ref, bfc_ref,
                out_ref):
    svals = p_ref[0, :_N, :] + p_ref[1, :_N, :] + g_ref[...]
    h = jnp.maximum(svals * dinv_ref[...] + b_ref[...], 0.0)
    gid = lax.broadcasted_iota(jnp.int32, (_G, _N), 0)
    onehot_t = (batch_ref[...] == gid).astype(jnp.float32)
    sums = jnp.dot(onehot_t, h,
                   preferred_element_type=jnp.float32, precision=_HI)
    cnt = jnp.sum(onehot_t, axis=1)[:, None]
    pooled = sums / jnp.maximum(cnt, 1.0)
    logit = jnp.dot(pooled, wfc_ref[...],
                    preferred_element_type=jnp.float32,
                    precision=_HI) + bfc_ref[...]
    out_ref[...] = jax.nn.sigmoid(logit)


def _final_call(p, g, dinv, b, batch2d, wfc, bfc):
    return pl.pallas_call(
        _final_body,
        out_shape=jax.ShapeDtypeStruct((_G, 1), jnp.float32),
    )(p, g, dinv, b, batch2d, wfc, bfc)


def kernel(x, edge_index, batch, W1, b1, W2, b2, W3, b3, Wfc, bfc):
    src = edge_index[0].astype(jnp.int32).reshape(_E // _K, _K)
    dst = edge_index[1].astype(jnp.int32).reshape(_E // _K, _K)
    batch2d = batch.astype(jnp.int32).reshape(1, _N)

    degp = _deg_call()(dst)
    g1, dinv = _b1_call(x, W1, degp)
    p1 = _make_edge_call(64)(g1, src, dst)
    g2 = _mid_call(p1, g1, dinv, b1.reshape(1, -1), W2)
    p2 = _make_edge_call(32)(g2, src, dst)
    g3 = _mid_call(p2, g2, dinv, b2.reshape(1, -1), W3)
    p3 = _make_edge_call(16)(g3, src, dst)
    out = _final_call(p3, g3, dinv, b3.reshape(1, -1), batch2d,
                      Wfc, bfc.reshape(1, 1))
    return out


# speedup vs baseline: 47.2113x; 1.0368x over previous
import functools

import jax
import jax.numpy as jnp
from jax import lax
from jax.experimental import pallas as pl
from jax.experimental.pallas import tpu as pltpu
from jax.experimental.pallas import tpu_sc as plsc

_N = 10000
_E = 320000
_G = 64

_NC = 2
_NS = 16
_NW = _NC * _NS
_EPW = _E // _NW
_K = 125
_NCHUNK = _EPW // _K
_GRP = 10
_NG = _NCHUNK // _GRP
_GRP_D = {64: 4, 32: 8, 16: 10}
_NP = 10240
_RPS = _NP // _NS
_ZR = 64
_DEGW = 16


@functools.lru_cache(maxsize=None)
def _sc_mesh():
    return plsc.VectorSubcoreMesh(core_axis_name="c", subcore_axis_name="s",
                                  num_cores=_NC, num_subcores=_NS)


def _zero_fill(zrow_v, d):
    z16 = jnp.zeros((16,), jnp.float32)
    for r in range(_ZR):
        for cc in range(d // 16):
            zrow_v[r, pl.ds(cc * 16, 16)] = z16


def _zero_accum(zrow_v, accum_sh, s):
    for j in range(_RPS // _ZR):
        pltpu.sync_copy(zrow_v, accum_sh.at[pl.ds(s * _RPS + j * _ZR, _ZR)])


def _deg_body(dst_hbm, out_hbm, didx_v, ones_v, zrow_v, ssem, accum_sh):
    c = lax.axis_index("c")
    s = lax.axis_index("s")
    one16 = jnp.ones((16,), jnp.float32)
    for r in range(_K):
        ones_v[r, :] = one16
    _zero_fill(zrow_v, _DEGW)
    _zero_accum(zrow_v, accum_sh, s)
    plsc.subcore_barrier()
    rbase = (s * _NC + c) * _NCHUNK
    pltpu.sync_copy(dst_hbm.at[pl.ds(rbase, _NCHUNK)], didx_v)

    def group(j, carry):
        descs = [pltpu.async_copy(ones_v, accum_sh.at[didx_v.at[_GRP * j + b]],
                                  ssem, add=True)
                 for b in range(_GRP)]
        for dsc in descs:
            dsc.wait()
        return carry

    lax.fori_loop(0, _NG, group, 0)
    plsc.subcore_barrier()
    pltpu.sync_copy(accum_sh.at[pl.ds(s * _RPS, _RPS)],
                    out_hbm.at[c, pl.ds(s * _RPS, _RPS)])


@functools.lru_cache(maxsize=None)
def _deg_call():
    return pl.kernel(
        _deg_body,
        out_type=jax.ShapeDtypeStruct((_NC, _NP, _DEGW), jnp.float32),
        mesh=_sc_mesh(),
        compiler_params=pltpu.CompilerParams(use_tc_tiling_on_sc=False),
        scratch_types=[
            pltpu.VMEM((_NCHUNK, _K), jnp.int32),
            pltpu.VMEM((_K, _DEGW), jnp.float32),
            pltpu.VMEM((_ZR, _DEGW), jnp.float32),
            pltpu.SemaphoreType.DMA,
            pltpu.VMEM_SHARED((_NP, _DEGW), jnp.float32),
        ],
    )


def _edge_body(d, g_hbm, src_hbm, dst_hbm, out_hbm,
               sidx_v, didx_v, rows_v, zrow_v, gsem0, gsem1, ssem0, ssem1,
               isem, accum_sh):
    grp = _GRP_D[d]
    ng = _NCHUNK // grp
    c = lax.axis_index("c")
    s = lax.axis_index("s")
    rbase = (s * _NC + c) * _NCHUNK
    di1 = pltpu.async_copy(src_hbm.at[pl.ds(rbase, _NCHUNK)], sidx_v, isem)
    di2 = pltpu.async_copy(dst_hbm.at[pl.ds(rbase, _NCHUNK)], didx_v, isem)
    _zero_fill(zrow_v, d)
    _zero_accum(zrow_v, accum_sh, s)
    di1.wait()
    di2.wait()
    plsc.subcore_barrier()

    def fire_gathers(g, bank, sem):
        for b in range(grp):
            pltpu.async_copy(g_hbm.at[sidx_v.at[grp * g + b]],
                             rows_v.at[bank, b], sem)

    def drain_gathers(bank, sem):
        for b in range(grp):
            pltpu.make_async_copy(g_hbm.at[sidx_v.at[b]],
                                  rows_v.at[bank, b], sem).wait()

    def fire_scatters(g, bank, sem):
        return [pltpu.async_copy(rows_v.at[bank, b],
                                 accum_sh.at[didx_v.at[grp * g + b]],
                                 sem, add=True)
                for b in range(grp)]

    fire_gathers(0, 0, gsem0)
    fire_gathers(1, 1, gsem1)

    def pipe(j, carry):
        drain_gathers(0, gsem0)
        s_a = fire_scatters(2 * j, 0, ssem0)
        drain_gathers(1, gsem1)
        for dsc in s_a:
            dsc.wait()

        @pl.when(j + 1 < ng // 2)
        def _():
            fire_gathers(2 * j + 2, 0, gsem0)

        s_b = fire_scatters(2 * j + 1, 1, ssem1)
        for dsc in s_b:
            dsc.wait()

        @pl.when(j + 1 < ng // 2)
        def _():
            fire_gathers(2 * j + 3, 1, gsem1)

        return carry

    lax.fori_loop(0, ng // 2, pipe, 0)
    plsc.subcore_barrier()
    pltpu.sync_copy(accum_sh.at[pl.ds(s * _RPS, _RPS)],
                    out_hbm.at[c, pl.ds(s * _RPS, _RPS)])


@functools.lru_cache(maxsize=None)
def _make_edge_call(d):
    return pl.kernel(
        functools.partial(_edge_body, d),
        out_type=jax.ShapeDtypeStruct((_NC, _NP, d), jnp.float32),
        mesh=_sc_mesh(),
        compiler_params=pltpu.CompilerParams(use_tc_tiling_on_sc=False),
        scratch_types=[
            pltpu.VMEM((_NCHUNK, _K), jnp.int32),
            pltpu.VMEM((_NCHUNK, _K), jnp.int32),
            pltpu.VMEM((2, _GRP_D[d], _K, d), jnp.float32),
            pltpu.VMEM((_ZR, d), jnp.float32),
            pltpu.SemaphoreType.DMA,
            pltpu.SemaphoreType.DMA,
            pltpu.SemaphoreType.DMA,
            pltpu.SemaphoreType.DMA,
            pltpu.SemaphoreType.DMA,
            pltpu.VMEM_SHARED((_NP, d), jnp.float32),
        ],
    )


_HI = lax.Precision.HIGHEST


def _b1_body(x_ref, w1_ref, degp_ref, g1_ref, dinv_ref):
    deg = (degp_ref[0, :_N, 0:1] + degp_ref[1, :_N, 0:1]) + 1.0
    dinv = lax.rsqrt(deg)
    h = jnp.dot(x_ref[...], w1_ref[...],
                preferred_element_type=jnp.float32, precision=_HI)
    g1_ref[...] = h * dinv
    dinv_ref[...] = dinv


def _b1_call(x, w1, degp):
    return pl.pallas_call(
        _b1_body,
        out_shape=(
            jax.ShapeDtypeStruct((_N, 64), jnp.float32),
            jax.ShapeDtypeStruct((_N, 1), jnp.float32),
        ),
    )(x, w1, degp)


def _mid_body(p_ref, g_ref, dinv_ref, b_ref, w_ref, out_ref):
    svals = p_ref[0, :_N, :] + p_ref[1, :_N, :] + g_ref[...]
    h = jnp.maximum(svals * dinv_ref[...] + b_ref[...], 0.0)
    out_ref[...] = jnp.dot(h, w_ref[...],
                           preferred_element_type=jnp.float32,
                           precision=_HI) * dinv_ref[...]


def _mid_call(p, g, dinv, b, w):
    return pl.pallas_call(
        _mid_body,
        out_shape=jax.ShapeDtypeStruct((_N, w.shape[1]), jnp.float32),
    )(p, g, dinv, b, w)


def _final_body(p_ref, g_ref, dinv_ref, b_ref, batch_ref, wfc_ref, bfc_ref,
                out_ref):
    svals = p_ref[0, :_N, :] + p_ref[1, :_N, :] + g_ref[...]
    h = jnp.maximum(svals * dinv_ref[...] + b_ref[...], 0.0)
    gid = lax.broadcasted_iota(jnp.int32, (_G, _N), 0)
    onehot_t = (batch_ref[...] == gid).astype(jnp.float32)
    sums = jnp.dot(onehot_t, h,
                   preferred_element_type=jnp.float32, precision=_HI)
    cnt = jnp.sum(onehot_t, axis=1)[:, None]
    pooled = sums / jnp.maximum(cnt, 1.0)
    logit = jnp.dot(pooled, wfc_ref[...],
                    preferred_element_type=jnp.float32,
                    precision=_HI) + bfc_ref[...]
    out_ref[...] = jax.nn.sigmoid(logit)


def _final_call(p, g, dinv, b, batch2d, wfc, bfc):
    return pl.pallas_call(
        _final_body,
        out_shape=jax.ShapeDtypeStruct((_G, 1), jnp.float32),
    )(p, g, dinv, b, batch2d, wfc, bfc)


def kernel(x, edge_index, batch, W1, b1, W2, b2, W3, b3, Wfc, bfc):
    src = edge_index[0].astype(jnp.int32).reshape(_E // _K, _K)
    dst = edge_index[1].astype(jnp.int32).reshape(_E // _K, _K)
    batch2d = batch.astype(jnp.int32).reshape(1, _N)

    degp = _deg_call()(dst)
    g1, dinv = _b1_call(x, W1, degp)
    p1 = _make_edge_call(64)(g1, src, dst)
    g2 = _mid_call(p1, g1, dinv, b1.reshape(1, -1), W2)
    p2 = _make_edge_call(32)(g2, src, dst)
    g3 = _mid_call(p2, g2, dinv, b2.reshape(1, -1), W3)
    p3 = _make_edge_call(16)(g3, src, dst)
    out = _final_call(p3, g3, dinv, b3.reshape(1, -1), batch2d,
                      Wfc, bfc.reshape(1, 1))
    return out


# speedup vs baseline: 47.2516x; 1.0009x over previous
import functools

import jax
import jax.numpy as jnp
from jax import lax
from jax.experimental import pallas as pl
from jax.experimental.pallas import tpu as pltpu
from jax.experimental.pallas import tpu_sc as plsc

_N = 10000
_E = 320000
_G = 64

_NC = 2
_NS = 16
_NW = _NC * _NS
_EPW = _E // _NW
_K = 125
_NCHUNK = _EPW // _K
_GRP = 10
_NG = _NCHUNK // _GRP
_GRP_D = {64: 4, 32: 8, 16: 10}
_NP = 10240
_RPS = _NP // _NS
_ZR = 64
_DEGW = 16


@functools.lru_cache(maxsize=None)
def _sc_mesh():
    return plsc.VectorSubcoreMesh(core_axis_name="c", subcore_axis_name="s",
                                  num_cores=_NC, num_subcores=_NS)


def _zero_fill(zrow_v, d):
    z16 = jnp.zeros((16,), jnp.float32)
    for r in range(_ZR):
        for cc in range(d // 16):
            zrow_v[r, pl.ds(cc * 16, 16)] = z16


def _zero_accum(zrow_v, accum_sh, s):
    for j in range(_RPS // _ZR):
        pltpu.sync_copy(zrow_v, accum_sh.at[pl.ds(s * _RPS + j * _ZR, _ZR)])


def _deg_body(dst_hbm, out_hbm, didx_v, ones_v, zrow_v, ssem, accum_sh):
    c = lax.axis_index("c")
    s = lax.axis_index("s")
    one16 = jnp.ones((16,), jnp.float32)
    for r in range(_K):
        ones_v[r, :] = one16
    _zero_fill(zrow_v, _DEGW)
    _zero_accum(zrow_v, accum_sh, s)
    plsc.subcore_barrier()
    rbase = (s * _NC + c) * _NCHUNK
    pltpu.sync_copy(dst_hbm.at[pl.ds(rbase, _NCHUNK)], didx_v)

    def group(j, carry):
        descs = [pltpu.async_copy(ones_v, accum_sh.at[didx_v.at[_GRP * j + b]],
                                  ssem, add=True)
                 for b in range(_GRP)]
        for dsc in descs:
            dsc.wait()
        return carry

    lax.fori_loop(0, _NG, group, 0)
    plsc.subcore_barrier()
    pltpu.sync_copy(accum_sh.at[pl.ds(s * _RPS, _RPS)],
                    out_hbm.at[c, pl.ds(s * _RPS, _RPS)])


@functools.lru_cache(maxsize=None)
def _deg_call():
    return pl.kernel(
        _deg_body,
        out_type=jax.ShapeDtypeStruct((_NC, _NP, _DEGW), jnp.float32),
        mesh=_sc_mesh(),
        compiler_params=pltpu.CompilerParams(use_tc_tiling_on_sc=False),
        scratch_types=[
            pltpu.VMEM((_NCHUNK, _K), jnp.int32),
            pltpu.VMEM((_K, _DEGW), jnp.float32),
            pltpu.VMEM((_ZR, _DEGW), jnp.float32),
            pltpu.SemaphoreType.DMA,
            pltpu.VMEM_SHARED((_NP, _DEGW), jnp.float32),
        ],
    )


def _edge_body(d, g_hbm, src_hbm, dst_hbm, out_hbm,
               sidx_v, didx_v, rows_v, zrow_v, gsem0, gsem1, ssem0, ssem1,
               isem, accum_sh):
    grp = _GRP_D[d]
    ng = _NCHUNK // grp
    c = lax.axis_index("c")
    s = lax.axis_index("s")
    rbase = (s * _NC + c) * _NCHUNK
    di1 = pltpu.async_copy(src_hbm.at[pl.ds(rbase, _NCHUNK)], sidx_v, isem)
    di2 = pltpu.async_copy(dst_hbm.at[pl.ds(rbase, _NCHUNK)], didx_v, isem)
    _zero_fill(zrow_v, d)
    _zero_accum(zrow_v, accum_sh, s)
    di1.wait()
    di2.wait()
    plsc.subcore_barrier()

    def fire_gathers(g, bank, sem):
        for b in range(grp):
            pltpu.async_copy(g_hbm.at[sidx_v.at[grp * g + b]],
                             rows_v.at[bank, b], sem)

    def drain_gathers(bank, sem):
        for b in range(grp):
            pltpu.make_async_copy(g_hbm.at[sidx_v.at[b]],
                                  rows_v.at[bank, b], sem).wait()

    def fire_scatters(g, bank, sem):
        return [pltpu.async_copy(rows_v.at[bank, b],
                                 accum_sh.at[didx_v.at[grp * g + b]],
                                 sem, add=True)
                for b in range(grp)]

    fire_gathers(0, 0, gsem0)
    fire_gathers(1, 1, gsem1)

    def pipe(j, carry):
        drain_gathers(0, gsem0)
        s_a = fire_scatters(2 * j, 0, ssem0)
        drain_gathers(1, gsem1)
        for dsc in s_a:
            dsc.wait()

        @pl.when(j + 1 < ng // 2)
        def _():
            fire_gathers(2 * j + 2, 0, gsem0)

        s_b = fire_scatters(2 * j + 1, 1, ssem1)
        for dsc in s_b:
            dsc.wait()

        @pl.when(j + 1 < ng // 2)
        def _():
            fire_gathers(2 * j + 3, 1, gsem1)

        return carry

    lax.fori_loop(0, ng // 2, pipe, 0)
    plsc.subcore_barrier()
    pltpu.sync_copy(accum_sh.at[pl.ds(s * _RPS, _RPS)],
                    out_hbm.at[c, pl.ds(s * _RPS, _RPS)])


@functools.lru_cache(maxsize=None)
def _make_edge_call(d):
    return pl.kernel(
        functools.partial(_edge_body, d),
        out_type=jax.ShapeDtypeStruct((_NC, _NP, d), jnp.float32),
        mesh=_sc_mesh(),
        compiler_params=pltpu.CompilerParams(use_tc_tiling_on_sc=False),
        scratch_types=[
            pltpu.VMEM((_NCHUNK, _K), jnp.int32),
            pltpu.VMEM((_NCHUNK, _K), jnp.int32),
            pltpu.VMEM((2, _GRP_D[d], _K, d), jnp.float32),
            pltpu.VMEM((_ZR, d), jnp.float32),
            pltpu.SemaphoreType.DMA,
            pltpu.SemaphoreType.DMA,
            pltpu.SemaphoreType.DMA,
            pltpu.SemaphoreType.DMA,
            pltpu.SemaphoreType.DMA,
            pltpu.VMEM_SHARED((_NP, d), jnp.float32),
        ],
    )


_HI = lax.Precision.HIGHEST


def _bm_body(x_ref, w1_ref, h_ref):
    h_ref[...] = jnp.dot(x_ref[...], w1_ref[...],
                         preferred_element_type=jnp.float32, precision=_HI)


def _bm_call(x, w1):
    return pl.pallas_call(
        _bm_body,
        out_shape=jax.ShapeDtypeStruct((_N, 64), jnp.float32),
    )(x, w1)


def _b1_body(h_ref, degp_ref, g1_ref, dinv_ref):
    deg = (degp_ref[0, :_N, 0:1] + degp_ref[1, :_N, 0:1]) + 1.0
    dinv = lax.rsqrt(deg)
    g1_ref[...] = h_ref[...] * dinv
    dinv_ref[...] = dinv


def _b1_call(h, degp):
    return pl.pallas_call(
        _b1_body,
        out_shape=(
            jax.ShapeDtypeStruct((_N, 64), jnp.float32),
            jax.ShapeDtypeStruct((_N, 1), jnp.float32),
        ),
    )(h, degp)


def _mid_body(p_ref, g_ref, dinv_ref, b_ref, w_ref, out_ref):
    svals = p_ref[0, :_N, :] + p_ref[1, :_N, :] + g_ref[...]
    h = jnp.maximum(svals * dinv_ref[...] + b_ref[...], 0.0)
    out_ref[...] = jnp.dot(h, w_ref[...],
                           preferred_element_type=jnp.float32,
                           precision=_HI) * dinv_ref[...]


def _mid_call(p, g, dinv, b, w):
    return pl.pallas_call(
        _mid_body,
        out_shape=jax.ShapeDtypeStruct((_N, w.shape[1]), jnp.float32),
    )(p, g, dinv, b, w)


def _final_body(p_ref, g_ref, dinv_ref, b_ref, batch_ref, wfc_ref, bfc_ref,
                out_ref):
    svals = p_ref[0, :_N, :] + p_ref[1, :_N, :] + g_ref[...]
    h = jnp.maximum(svals * dinv_ref[...] + b_ref[...], 0.0)
    gid = lax.broadcasted_iota(jnp.int32, (_G, _N), 0)
    onehot_t = (batch_ref[...] == gid).astype(jnp.float32)
    sums = jnp.dot(onehot_t, h,
                   preferred_element_type=jnp.float32, precision=_HI)
    cnt = jnp.sum(onehot_t, axis=1)[:, None]
    pooled = sums / jnp.maximum(cnt, 1.0)
    logit = jnp.dot(pooled, wfc_ref[...],
                    preferred_element_type=jnp.float32,
                    precision=_HI) + bfc_ref[...]
    out_ref[...] = jax.nn.sigmoid(logit)


def _final_call(p, g, dinv, b, batch2d, wfc, bfc):
    return pl.pallas_call(
        _final_body,
        out_shape=jax.ShapeDtypeStruct((_G, 1), jnp.float32),
    )(p, g, dinv, b, batch2d, wfc, bfc)


def kernel(x, edge_index, batch, W1, b1, W2, b2, W3, b3, Wfc, bfc):
    src = edge_index[0].astype(jnp.int32).reshape(_E // _K, _K)
    dst = edge_index[1].astype(jnp.int32).reshape(_E // _K, _K)
    batch2d = batch.astype(jnp.int32).reshape(1, _N)

    degp = _deg_call()(dst)
    h1 = _bm_call(x, W1)
    g1, dinv = _b1_call(h1, degp)
    p1 = _make_edge_call(64)(g1, src, dst)
    g2 = _mid_call(p1, g1, dinv, b1.reshape(1, -1), W2)
    p2 = _make_edge_call(32)(g2, src, dst)
    g3 = _mid_call(p2, g2, dinv, b2.reshape(1, -1), W3)
    p3 = _make_edge_call(16)(g3, src, dst)
    out = _final_call(p3, g3, dinv, b3.reshape(1, -1), batch2d,
                      Wfc, bfc.reshape(1, 1))
    return out
